# Initial kernel scaffold; baseline (speedup 1.0000x reference)
import jax
import jax.numpy as jnp
from jax.experimental import pallas as pl


def kernel(x, atom_embedding_weight):
    raise NotImplementedError("write your pallas kernel here")


# speedup vs baseline: 1.0592x; 1.0592x over previous
import functools

import jax
import jax.numpy as jnp
from jax import lax
from jax.experimental import pallas as pl
from jax.experimental.pallas import tpu as pltpu
from jax.experimental.pallas import tpu_sc as plsc

N_NODES = 100000
N_FEATS = 9
EMB_DIM = 512

_INFO = plsc.get_sparse_core_info()
NC = _INFO.num_cores
NS = _INFO.num_subcores
L = _INFO.num_lanes
NW = NC * NS

WINDOW = 3200
CHUNK = 128
N_CHUNKS = WINDOW // CHUNK
N_EXTRACT = WINDOW // L


def _sc_body(x_hbm, table_hbm, out_hbm, xwin_v, idx_v, rows_v, sem):
    wid = lax.axis_index("s") * NC + lax.axis_index("c")
    start = jnp.minimum(wid * WINDOW, N_NODES - WINDOW)

    pltpu.sync_copy(x_hbm.at[pl.ds(start * N_FEATS, WINDOW * N_FEATS)], xwin_v)

    def extract(j, _):
        flat16 = (j * L + lax.iota(jnp.int32, L)) * N_FEATS
        vals = plsc.load_gather(xwin_v, [flat16])
        idx_v[pl.ds(j * L, L)] = vals
        return _

    lax.fori_loop(0, N_EXTRACT, extract, None)

    def chunk(k, _):
        off = k * CHUNK
        pltpu.async_copy(
            table_hbm.at[idx_v.at[pl.ds(off, CHUNK)]], rows_v, sem
        ).wait()
        pltpu.sync_copy(rows_v, out_hbm.at[pl.ds(start + off, CHUNK)])
        return _

    lax.fori_loop(0, N_CHUNKS, chunk, None)


@jax.jit
def kernel(x, atom_embedding_weight):
    mesh = plsc.VectorSubcoreMesh(core_axis_name="c", subcore_axis_name="s")
    run = functools.partial(
        pl.kernel,
        mesh=mesh,
        out_type=jax.ShapeDtypeStruct((N_NODES, EMB_DIM), jnp.float32),
        scratch_types=[
            pltpu.VMEM((WINDOW * N_FEATS,), jnp.int32),
            pltpu.VMEM((WINDOW,), jnp.int32),
            pltpu.VMEM((CHUNK, EMB_DIM), jnp.float32),
            pltpu.SemaphoreType.DMA,
        ],
        compiler_params=pltpu.CompilerParams(needs_layout_passes=False),
    )(_sc_body)
    return run(x.reshape(-1), atom_embedding_weight)


# speedup vs baseline: 1.0629x; 1.0035x over previous
import functools

import jax
import jax.numpy as jnp
from jax import lax
from jax.experimental import pallas as pl
from jax.experimental.pallas import tpu as pltpu
from jax.experimental.pallas import tpu_sc as plsc

N_NODES = 100000
N_FEATS = 9
EMB_DIM = 512

_INFO = plsc.get_sparse_core_info()
NC = _INFO.num_cores
NS = _INFO.num_subcores
L = _INFO.num_lanes
NW = NC * NS

WINDOW = 3200
CHUNK = 64
N_CHUNKS = WINDOW // CHUNK
N_EXTRACT = WINDOW // L


def _sc_body(x_hbm, table_hbm, out_hbm, xwin_v, idx_v, rows_v, sem):
    wid = lax.axis_index("s") * NC + lax.axis_index("c")
    start = jnp.minimum(wid * WINDOW, N_NODES - WINDOW)

    pltpu.sync_copy(x_hbm.at[pl.ds(start * N_FEATS, WINDOW * N_FEATS)], xwin_v)

    def extract(j, _):
        flat16 = (j * L + lax.iota(jnp.int32, L)) * N_FEATS
        vals = plsc.load_gather(xwin_v, [flat16])
        idx_v[pl.ds(j * L, L)] = vals
        return _

    lax.fori_loop(0, N_EXTRACT, extract, None)

    def start_gather(k, buf):
        off = k * CHUNK
        pltpu.async_copy(
            table_hbm.at[idx_v.at[pl.ds(off, CHUNK)]],
            rows_v.at[buf],
            sem.at[buf],
        )

    start_gather(0, 0)

    def chunk(k, _):
        b = lax.rem(k, 2)
        nb = 1 - b

        @pl.when(k + 1 < N_CHUNKS)
        def _():
            start_gather(k + 1, nb)

        off = k * CHUNK
        pltpu.make_async_copy(
            table_hbm.at[idx_v.at[pl.ds(off, CHUNK)]],
            rows_v.at[b],
            sem.at[b],
        ).wait()
        pltpu.sync_copy(rows_v.at[b], out_hbm.at[pl.ds(start + off, CHUNK)])
        return _

    lax.fori_loop(0, N_CHUNKS, chunk, None)


@jax.jit
def kernel(x, atom_embedding_weight):
    mesh = plsc.VectorSubcoreMesh(core_axis_name="c", subcore_axis_name="s")
    run = functools.partial(
        pl.kernel,
        mesh=mesh,
        out_type=jax.ShapeDtypeStruct((N_NODES, EMB_DIM), jnp.float32),
        scratch_types=[
            pltpu.VMEM((WINDOW * N_FEATS,), jnp.int32),
            pltpu.VMEM((WINDOW,), jnp.int32),
            pltpu.VMEM((2, CHUNK, EMB_DIM), jnp.float32),
            pltpu.SemaphoreType.DMA((2,)),
        ],
        compiler_params=pltpu.CompilerParams(needs_layout_passes=False),
    )(_sc_body)
    return run(x.reshape(-1), atom_embedding_weight)


# speedup vs baseline: 1.5146x; 1.4250x over previous
import functools

import jax
import jax.numpy as jnp
from jax import lax
from jax.experimental import pallas as pl
from jax.experimental.pallas import tpu as pltpu
from jax.experimental.pallas import tpu_sc as plsc

N_NODES = 100000
N_FEATS = 9
EMB_DIM = 512

_INFO = plsc.get_sparse_core_info()
NC = _INFO.num_cores
NS = _INFO.num_subcores
L = _INFO.num_lanes
NW = NC * NS

NUM_EMB = 119
N_REP = 8
WINDOW = 3200
CHUNK = 64
N_CHUNKS = WINDOW // CHUNK
N_EXTRACT = WINDOW // L


def _sc_body(x_hbm, table_hbm, out_hbm, xwin_v, idx_v, rows_v, sem):
    wid = lax.axis_index("s") * NC + lax.axis_index("c")
    start = jnp.minimum(wid * WINDOW, N_NODES - WINDOW)

    rep_off = lax.rem(wid, N_REP) * NUM_EMB

    pltpu.sync_copy(x_hbm.at[pl.ds(start * N_FEATS, WINDOW * N_FEATS)], xwin_v)

    def extract(j, _):
        flat16 = (j * L + lax.iota(jnp.int32, L)) * N_FEATS
        vals = plsc.load_gather(xwin_v, [flat16]) + rep_off
        idx_v[lax.div(j, CHUNK // L), pl.ds(lax.rem(j, CHUNK // L) * L, L)] = (
            vals
        )
        return _

    lax.fori_loop(0, N_EXTRACT, extract, None)

    def start_gather(k, buf):
        off = k * CHUNK
        pltpu.async_copy(
            table_hbm.at[idx_v.at[k]],
            rows_v.at[buf],
            sem.at[buf],
        )

    start_gather(0, 0)

    def chunk(k, _):
        b = lax.rem(k, 2)
        nb = 1 - b

        @pl.when(k + 1 < N_CHUNKS)
        def _():
            start_gather(k + 1, nb)

        pltpu.make_async_copy(
            table_hbm.at[idx_v.at[k]],
            rows_v.at[b],
            sem.at[b],
        ).wait()
        off = k * CHUNK
        pltpu.sync_copy(rows_v.at[b], out_hbm.at[pl.ds(start + off, CHUNK)])
        return _

    lax.fori_loop(0, N_CHUNKS, chunk, None)


@jax.jit
def kernel(x, atom_embedding_weight):
    mesh = plsc.VectorSubcoreMesh(core_axis_name="c", subcore_axis_name="s")
    run = functools.partial(
        pl.kernel,
        mesh=mesh,
        out_type=jax.ShapeDtypeStruct((N_NODES, EMB_DIM), jnp.float32),
        scratch_types=[
            pltpu.VMEM((WINDOW * N_FEATS,), jnp.int32),
            pltpu.VMEM((N_CHUNKS, CHUNK), jnp.int32),
            pltpu.VMEM((2, CHUNK, EMB_DIM), jnp.float32),
            pltpu.SemaphoreType.DMA((2,)),
        ],
        compiler_params=pltpu.CompilerParams(needs_layout_passes=False),
    )(_sc_body)
    table_rep = jnp.tile(atom_embedding_weight, (N_REP, 1))
    return run(x.reshape(-1), table_rep)


# speedup vs baseline: 1.5738x; 1.0391x over previous
import functools

import jax
import jax.numpy as jnp
from jax import lax
from jax.experimental import pallas as pl
from jax.experimental.pallas import tpu as pltpu
from jax.experimental.pallas import tpu_sc as plsc

N_NODES = 100000
N_FEATS = 9
EMB_DIM = 512

_INFO = plsc.get_sparse_core_info()
NC = _INFO.num_cores
NS = _INFO.num_subcores
L = _INFO.num_lanes
NW = NC * NS

NUM_EMB = 119
N_REP = 32
WINDOW = 3200
CHUNK = 64
N_CHUNKS = WINDOW // CHUNK
N_EXTRACT = WINDOW // L


def _sc_body(x_hbm, table_hbm, out_hbm, xwin_v, idx_v, rows_v, sem):
    wid = lax.axis_index("s") * NC + lax.axis_index("c")
    start = jnp.minimum(wid * WINDOW, N_NODES - WINDOW)

    rep_off = lax.rem(wid, N_REP) * NUM_EMB

    pltpu.sync_copy(x_hbm.at[pl.ds(start * N_FEATS, WINDOW * N_FEATS)], xwin_v)

    def extract(j, _):
        flat16 = (j * L + lax.iota(jnp.int32, L)) * N_FEATS
        vals = plsc.load_gather(xwin_v, [flat16]) + rep_off
        idx_v[lax.div(j, CHUNK // L), pl.ds(lax.rem(j, CHUNK // L) * L, L)] = (
            vals
        )
        return _

    lax.fori_loop(0, N_EXTRACT, extract, None)

    def start_gather(k, buf):
        off = k * CHUNK
        pltpu.async_copy(
            table_hbm.at[idx_v.at[k]],
            rows_v.at[buf],
            sem.at[buf],
        )

    start_gather(0, 0)

    def chunk(k, _):
        b = lax.rem(k, 2)
        nb = 1 - b

        @pl.when(k + 1 < N_CHUNKS)
        def _():
            start_gather(k + 1, nb)

        pltpu.make_async_copy(
            table_hbm.at[idx_v.at[k]],
            rows_v.at[b],
            sem.at[b],
        ).wait()
        off = k * CHUNK
        pltpu.sync_copy(rows_v.at[b], out_hbm.at[pl.ds(start + off, CHUNK)])
        return _

    lax.fori_loop(0, N_CHUNKS, chunk, None)


@jax.jit
def kernel(x, atom_embedding_weight):
    mesh = plsc.VectorSubcoreMesh(core_axis_name="c", subcore_axis_name="s")
    run = functools.partial(
        pl.kernel,
        mesh=mesh,
        out_type=jax.ShapeDtypeStruct((N_NODES, EMB_DIM), jnp.float32),
        scratch_types=[
            pltpu.VMEM((WINDOW * N_FEATS,), jnp.int32),
            pltpu.VMEM((N_CHUNKS, CHUNK), jnp.int32),
            pltpu.VMEM((2, CHUNK, EMB_DIM), jnp.float32),
            pltpu.SemaphoreType.DMA((2,)),
        ],
        compiler_params=pltpu.CompilerParams(needs_layout_passes=False),
    )(_sc_body)
    table_rep = jnp.tile(atom_embedding_weight, (N_REP, 1))
    return run(x.reshape(-1), table_rep)
